# baseline (speedup 1.0000x reference)
import functools

import jax
import jax.numpy as jnp
from jax import lax
from jax.experimental import pallas as pl
from jax.experimental.pallas import tpu as pltpu
from jax.experimental.pallas import tpu_sc as plsc

D = 128
L = 50
B = 1024
_NC = 2
_NS = 16
_NW = _NC * _NS
_N_IDX = B * L
_BPW = _N_IDX // _NW
_C = 80
_NCHUNK = _BPW // _C


def _sc_gather(dx_table, rx_table, lab_table, dx_idx, md_idx, lb_idx):
    mesh = plsc.VectorSubcoreMesh(core_axis_name="c", subcore_axis_name="s")
    out_t = [jax.ShapeDtypeStruct((_N_IDX, D), jnp.float32)] * 3

    @functools.partial(
        pl.kernel,
        mesh=mesh,
        out_type=out_t,
        scratch_types=[
            pltpu.VMEM((_C,), jnp.int32),
            pltpu.VMEM((_C, D), jnp.float32),
            pltpu.SemaphoreType.DMA,
        ],
    )
    def gather_kernel(dx_t, rx_t, lb_t, dxi, mdi, lbi, o_dx, o_md, o_lb,
                      idx_v, rows_v, sem):
        wid = lax.axis_index("s") * _NC + lax.axis_index("c")
        base = wid * _BPW

        def one_table(tab, idx_hbm, out_hbm):
            def body(j, carry):
                off = pl.multiple_of(base + j * _C, 8)
                pltpu.sync_copy(idx_hbm.at[pl.ds(off, _C)], idx_v)
                pltpu.async_copy(tab.at[idx_v], rows_v, sem).wait()
                pltpu.sync_copy(rows_v, out_hbm.at[pl.ds(off, _C)])
                return carry
            lax.fori_loop(0, _NCHUNK, body, 0)

        one_table(dx_t, dxi, o_dx)
        one_table(rx_t, mdi, o_md)
        one_table(lb_t, lbi, o_lb)

    return gather_kernel(dx_table, rx_table, lab_table, dx_idx, md_idx, lb_idx)


def _tc_body(dxg, dxt, dxm, mdg, mdt, mdm, lbg, lbt, lbm, lbv, dm, dse,
             wd, bd_, wp, bp_, w1_, b1_, w2_, b2_, g_, bt_,
             o_dm, o_dx, o_md, o_lb, o_ds, *, bb):
    half = lax.broadcasted_iota(jnp.int32, (1, 1, D // 2), 2).astype(jnp.float32)
    div = jnp.exp(half * (-2.0 * jnp.log(10000.0) / D))
    pos = lax.broadcasted_iota(jnp.int32, (1, L, 1), 1).astype(jnp.float32)
    pe = jnp.concatenate([jnp.sin(pos * div), jnp.cos(pos * div)], axis=-1)

    gm = g_[...].reshape(1, 1, D)
    bt = bt_[...].reshape(1, 1, D)

    def ln3(e):
        mu = jnp.mean(e, axis=-1, keepdims=True)
        var = jnp.mean((e - mu) ** 2, axis=-1, keepdims=True)
        return (e - mu) * lax.rsqrt(var + 1e-5) * gm + bt

    def time_enc(t):
        ang = t[:, :, None] * div
        return jnp.concatenate([jnp.sin(ang), jnp.cos(ang)], axis=-1)

    def path(rows, t, m):
        return ln3(rows + time_enc(t) + pe) * m[:, :, None]

    o_dx[...] = path(dxg[...], dxt[...], dxm[...])
    o_md[...] = path(mdg[...], mdt[...], mdm[...])

    h = jnp.maximum(
        lbv[...][:, :, None] * w1_[...].reshape(1, 1, D // 2)
        + b1_[...].reshape(1, 1, D // 2), 0.0)
    v = jnp.dot(h.reshape(bb * L, D // 2), w2_[...],
                preferred_element_type=jnp.float32).reshape(bb, L, D)
    v = v + b2_[...].reshape(1, 1, D)
    o_lb[...] = ln3(lbg[...] + v + time_enc(lbt[...]) + pe) * lbm[...][:, :, None]

    o_dm[...] = jnp.dot(dm[...], wd[...],
                        preferred_element_type=jnp.float32) + bd_[...]

    x = (dse[...][:, 0, :] + dse[...][:, 1, :]) * 0.5
    y = jnp.dot(x, wp[...], preferred_element_type=jnp.float32) + bp_[...]
    mu = jnp.mean(y, axis=-1, keepdims=True)
    var = jnp.mean((y - mu) ** 2, axis=-1, keepdims=True)
    o_ds[...] = (y - mu) * lax.rsqrt(var + 1e-5) * g_[...] + bt_[...]


def _tc_fused(dx_rows, md_rows, lb_rows, dx_times, dx_mask, med_times, med_mask,
              lab_times, lab_mask, lab_vals, demographic, ds_emb,
              Wd, bd, Wp, bp, w1, b1, W2, b2, gamma, beta):
    bb = 64
    grid = (B // bb,)

    def blk(shape):
        return pl.BlockSpec(shape, lambda i: (i,) + (0,) * (len(shape) - 1))

    def full(shape):
        return pl.BlockSpec(shape, lambda i: (0,) * len(shape))

    f32 = jnp.float32
    return pl.pallas_call(
        functools.partial(_tc_body, bb=bb),
        grid=grid,
        in_specs=[
            blk((bb, L, D)), blk((bb, L)), blk((bb, L)),
            blk((bb, L, D)), blk((bb, L)), blk((bb, L)),
            blk((bb, L, D)), blk((bb, L)), blk((bb, L)), blk((bb, L)),
            blk((bb, 70)), blk((bb, 2, 768)),
            full((70, D)), full((1, D)), full((768, D)), full((1, D)),
            full((1, D // 2)), full((1, D // 2)), full((D // 2, D)),
            full((1, D)), full((1, D)), full((1, D)),
        ],
        out_specs=[
            blk((bb, D)), blk((bb, L, D)), blk((bb, L, D)), blk((bb, L, D)),
            blk((bb, D)),
        ],
        out_shape=[
            jax.ShapeDtypeStruct((B, D), f32),
            jax.ShapeDtypeStruct((B, L, D), f32),
            jax.ShapeDtypeStruct((B, L, D), f32),
            jax.ShapeDtypeStruct((B, L, D), f32),
            jax.ShapeDtypeStruct((B, D), f32),
        ],
    )(dx_rows, dx_times, dx_mask, md_rows, med_times, med_mask,
      lb_rows, lab_times, lab_mask, lab_vals, demographic, ds_emb,
      Wd, bd, Wp, bp, w1, b1, W2, b2, gamma, beta)


def kernel(demographic, dx_codes, dx_times, dx_mask, med_codes, med_times,
           med_mask, lab_codes, lab_times, lab_values, lab_mask, ds_emb,
           dx_table, rx_table, lab_table, Wd, bd, Wp, bp, Wv1, bv1, Wv2, bv2,
           gamma, beta):
    dxi = dx_codes.reshape(-1).astype(jnp.int32)
    mdi = med_codes.reshape(-1).astype(jnp.int32)
    lbi = lab_codes.reshape(-1).astype(jnp.int32)

    dxr, mdr, lbr = _sc_gather(dx_table, rx_table, lab_table, dxi, mdi, lbi)

    o_dm, o_dx, o_md, o_lb, o_ds = _tc_fused(
        dxr.reshape(B, L, D), mdr.reshape(B, L, D), lbr.reshape(B, L, D),
        dx_times, dx_mask, med_times, med_mask,
        lab_times, lab_mask, lab_values.reshape(B, L),
        demographic, ds_emb,
        Wd, bd.reshape(1, D), Wp, bp.reshape(1, D),
        Wv1.reshape(1, D // 2), bv1.reshape(1, D // 2),
        Wv2, bv2.reshape(1, D), gamma.reshape(1, D), beta.reshape(1, D))
    return (o_dm, o_dx, o_md, o_lb, o_ds)
